# baseline (speedup 1.0000x reference)
import functools

import jax
import jax.numpy as jnp
from jax import lax
from jax.experimental import pallas as pl
from jax.experimental.pallas import tpu as pltpu
from jax.experimental.pallas import tpu_sc as plsc

N = 10000
D = 256
HD = 128
E = 160000
EL = 50000

NSUB = 16
NCORE = 2

ECHUNK = 128
CPT = 80
EPAD = NSUB * CPT * ECHUNK
ACC_R = 10112
ZR = ACC_R // NSUB
OW = 624
OTAIL = N - NSUB * OW
CW = 128
CWI = 16

DCPT = 16
DW = 26
ELPAD = DW * DCPT * ECHUNK

_mesh = plsc.VectorSubcoreMesh(core_axis_name="c", subcore_axis_name="s")


HCPT = CPT // 2


def _seg_body(xt2, srclo, srchi, dstp, z_a, sums,
              acc, idxs, idxd, rows0, rows1, sem0, sem1, ssem0, ssem1):
    c = lax.axis_index("c")
    s = lax.axis_index("s")

    zoff = pl.multiple_of(s * ZR, 8)
    coff = pl.multiple_of(s * CPT, 8)
    pltpu.sync_copy(z_a, acc.at[pl.ds(zoff, ZR)])

    plsc.subcore_barrier()

    for h in range(2):
        pltpu.sync_copy(dstp.at[pl.ds(coff + h * HCPT, HCPT)], idxd)

        @pl.when(c == 0)
        def _():
            pltpu.sync_copy(srclo.at[pl.ds(coff + h * HCPT, HCPT)], idxs)

        @pl.when(c == 1)
        def _():
            pltpu.sync_copy(srchi.at[pl.ds(coff + h * HCPT, HCPT)], idxs)

        pltpu.async_copy(xt2.at[idxs.at[0]], rows0, sem0)
        pltpu.async_copy(xt2.at[idxs.at[1]], rows1, sem1)

        def pair(g, carry):
            j0 = 2 * g
            j1 = j0 + 1
            pltpu.make_async_copy(xt2.at[idxs.at[j0]], rows0, sem0).wait()
            pltpu.async_copy(rows0, acc.at[idxd.at[j0]], ssem0, add=True)
            pltpu.make_async_copy(xt2.at[idxs.at[j1]], rows1, sem1).wait()
            pltpu.async_copy(rows1, acc.at[idxd.at[j1]], ssem1, add=True)
            pltpu.make_async_copy(rows0, acc.at[idxd.at[j0]], ssem0).wait()

            @pl.when(j0 + 2 < HCPT)
            def _():
                pltpu.async_copy(xt2.at[idxs.at[j0 + 2]], rows0, sem0)

            pltpu.make_async_copy(rows1, acc.at[idxd.at[j1]], ssem1).wait()

            @pl.when(j1 + 2 < HCPT)
            def _():
                pltpu.async_copy(xt2.at[idxs.at[j1 + 2]], rows1, sem1)

            return carry

        lax.fori_loop(0, HCPT // 2, pair, 0)

    plsc.subcore_barrier()

    pltpu.sync_copy(acc.at[pl.ds(pl.multiple_of(s * OW, 8), OW)],
                    sums.at[pl.ds(pl.multiple_of(c * N + s * OW, 8), OW)])

    @pl.when(s == NSUB - 1)
    def _():
        pltpu.sync_copy(
            acc.at[pl.ds(NSUB * OW, OTAIL)],
            sums.at[pl.ds(pl.multiple_of(c * N + NSUB * OW, 8), OTAIL)])


_seg = pl.kernel(
    _seg_body,
    out_type=jax.ShapeDtypeStruct((NCORE * N, HD), jnp.float32),
    mesh=_mesh,
    scratch_types=[
        pltpu.VMEM_SHARED((ACC_R, HD), jnp.float32),
        pltpu.VMEM((HCPT, ECHUNK), jnp.int32),
        pltpu.VMEM((HCPT, ECHUNK), jnp.int32),
        pltpu.VMEM((ECHUNK, HD), jnp.float32),
        pltpu.VMEM((ECHUNK, HD), jnp.float32),
        pltpu.SemaphoreType.DMA, pltpu.SemaphoreType.DMA,
        pltpu.SemaphoreType.DMA, pltpu.SemaphoreType.DMA,
    ],
)


def _cnt_body(dst_a, dst_b, z_c8, o_c8, cnt_a, cnt_b, cnta, idxd, onesb,
              csem):
    c = lax.axis_index("c")
    s = lax.axis_index("s")
    zoff = pl.multiple_of(s * ZR, 8)
    coff = pl.multiple_of(s * CPT, 8)
    pltpu.sync_copy(z_c8, cnta.at[pl.ds(zoff, ZR)])
    pltpu.sync_copy(o_c8, onesb)

    plsc.subcore_barrier()

    for h in range(2):
        @pl.when(c == 0)
        def _():
            pltpu.sync_copy(dst_a.at[pl.ds(coff + h * HCPT, HCPT)], idxd)

        @pl.when(c == 1)
        def _():
            pltpu.sync_copy(dst_b.at[pl.ds(coff + h * HCPT, HCPT)], idxd)

        def grp(gi, carry):
            for k in range(8):
                pltpu.async_copy(onesb, cnta.at[idxd.at[gi * 8 + k]], csem,
                                 add=True)
            for k in range(8):
                pltpu.make_async_copy(onesb, cnta.at[idxd.at[gi * 8 + k]],
                                      csem).wait()
            return carry

        lax.fori_loop(0, HCPT // 8, grp, 0)

    plsc.subcore_barrier()

    woff = pl.multiple_of(s * OW, 8)

    @pl.when(c == 0)
    def _():
        pltpu.sync_copy(cnta.at[pl.ds(woff, OW)], cnt_a.at[pl.ds(woff, OW)])

        @pl.when(s == NSUB - 1)
        def _():
            pltpu.sync_copy(cnta.at[pl.ds(NSUB * OW, OTAIL)],
                            cnt_a.at[pl.ds(NSUB * OW, OTAIL)])

    @pl.when(c == 1)
    def _():
        pltpu.sync_copy(cnta.at[pl.ds(woff, OW)], cnt_b.at[pl.ds(woff, OW)])

        @pl.when(s == NSUB - 1)
        def _():
            pltpu.sync_copy(cnta.at[pl.ds(NSUB * OW, OTAIL)],
                            cnt_b.at[pl.ds(NSUB * OW, OTAIL)])


_cnt = pl.kernel(
    _cnt_body,
    out_type=(jax.ShapeDtypeStruct((N, CW), jnp.float32),
              jax.ShapeDtypeStruct((N, CW), jnp.float32)),
    mesh=_mesh,
    scratch_types=[
        pltpu.VMEM_SHARED((ACC_R, CW), jnp.float32),
        pltpu.VMEM((HCPT, ECHUNK), jnp.int32),
        pltpu.VMEM((ECHUNK, CW), jnp.float32),
        pltpu.SemaphoreType.DMA,
    ],
)


def _dec_gather_body(zc2, zv2, rowlo, rowhi, collo, colhi, gzc, gzv,
                     idxrl, idxrh, idxcl, idxch, bufa, bufb, bufc, bufd,
                     gsa, gsb, gsc, gsd, tsa, tsb, tsc, tsd):
    c = lax.axis_index("c")
    s = lax.axis_index("s")
    w = s * NCORE + c

    @pl.when(w < DW)
    def _():
        soff = pl.multiple_of(w * DCPT, 8)
        pltpu.sync_copy(rowlo.at[pl.ds(soff, DCPT)], idxrl)
        pltpu.sync_copy(rowhi.at[pl.ds(soff, DCPT)], idxrh)
        pltpu.sync_copy(collo.at[pl.ds(soff, DCPT)], idxcl)
        pltpu.sync_copy(colhi.at[pl.ds(soff, DCPT)], idxch)

        def stream(i):
            tbl = (zc2, zc2, zv2, zv2)[i]
            idx = (idxrl, idxrh, idxcl, idxch)[i]
            out = (gzc, gzc, gzv, gzv)[i]
            base = (0, ELPAD, 0, ELPAD)[i]
            buf = (bufa, bufb, bufc, bufd)[i]
            gs = (gsa, gsb, gsc, gsd)[i]
            ts = (tsa, tsb, tsc, tsd)[i]
            return tbl, idx, out, base, buf, gs, ts

        for i in range(4):
            tbl, idx, out, base, buf, gs, ts = stream(i)
            pltpu.async_copy(tbl.at[idx.at[0]], buf, gs)

        def jbody(j, carry):
            o = pl.multiple_of((w * DCPT + j) * ECHUNK, 8)
            for i in range(4):
                tbl, idx, out, base, buf, gs, ts = stream(i)
                pltpu.make_async_copy(tbl.at[idx.at[j]], buf, gs).wait()
                pltpu.async_copy(buf, out.at[pl.ds(base + o, ECHUNK)], ts)
            for i in range(4):
                tbl, idx, out, base, buf, gs, ts = stream(i)
                pltpu.make_async_copy(buf, out.at[pl.ds(base + o, ECHUNK)],
                                      ts).wait()

                @pl.when(j + 1 < DCPT)
                def _():
                    pltpu.async_copy(tbl.at[idx.at[j + 1]], buf, gs)

            return carry

        lax.fori_loop(0, DCPT, jbody, 0)


_dec_gather = pl.kernel(
    _dec_gather_body,
    out_type=(jax.ShapeDtypeStruct((2 * ELPAD, HD), jnp.float32),
              jax.ShapeDtypeStruct((2 * ELPAD, HD), jnp.float32)),
    mesh=_mesh,
    scratch_types=[
        pltpu.VMEM((DCPT, ECHUNK), jnp.int32),
        pltpu.VMEM((DCPT, ECHUNK), jnp.int32),
        pltpu.VMEM((DCPT, ECHUNK), jnp.int32),
        pltpu.VMEM((DCPT, ECHUNK), jnp.int32),
        pltpu.VMEM((ECHUNK, HD), jnp.float32),
        pltpu.VMEM((ECHUNK, HD), jnp.float32),
        pltpu.VMEM((ECHUNK, HD), jnp.float32),
        pltpu.VMEM((ECHUNK, HD), jnp.float32),
        pltpu.SemaphoreType.DMA, pltpu.SemaphoreType.DMA,
        pltpu.SemaphoreType.DMA, pltpu.SemaphoreType.DMA,
        pltpu.SemaphoreType.DMA, pltpu.SemaphoreType.DMA,
        pltpu.SemaphoreType.DMA, pltpu.SemaphoreType.DMA,
    ],
)


_BM = 1000


def _combine_body(s2, cntr, x, wl, wr, b, o, o2=None, *, relu, x_split,
                  o_split):
    s = jnp.concatenate([s2[0], s2[1]], axis=-1)
    mean = s / jnp.maximum(cntr[:, 0], 1.0)[:, None]
    if x_split:
        xv = jnp.concatenate([x[0], x[1]], axis=-1)
    else:
        xv = x[...]
    r = (jnp.dot(mean, wl[...], preferred_element_type=jnp.float32)
         + jnp.dot(xv, wr[...], preferred_element_type=jnp.float32)
         + b[...])
    if relu:
        r = jnp.maximum(r, 0.0)
    if o_split:
        o[0] = r[:, :HD]
        o[1] = r[:, HD:]
    else:
        o[...] = r
        o2[0] = r[:, :HD]
        o2[1] = r[:, HD:]


def _make_combine(relu, x_split, o_split):
    split3 = pl.BlockSpec((2, _BM, HD), lambda i: (0, i, 0))
    full2 = pl.BlockSpec((_BM, D), lambda i: (i, 0))
    wspec = pl.BlockSpec((D, D), lambda i: (0, 0))
    return pl.pallas_call(
        functools.partial(_combine_body, relu=relu, x_split=x_split,
                          o_split=o_split),
        grid=(N // _BM,),
        in_specs=[
            split3,
            pl.BlockSpec((_BM, CW), lambda i: (i, 0)),
            split3 if x_split else full2,
            wspec, wspec,
            pl.BlockSpec((1, D), lambda i: (0, 0)),
        ],
        out_specs=split3 if o_split else [full2, split3],
        out_shape=jax.ShapeDtypeStruct((2, N, HD), jnp.float32)
        if o_split else [jax.ShapeDtypeStruct((N, D), jnp.float32),
                         jax.ShapeDtypeStruct((2, N, HD), jnp.float32)],
    )


_combine_l1 = _make_combine(relu=True, x_split=False, o_split=True)
_combine_l2 = _make_combine(relu=False, x_split=True, o_split=False)


_DBM = 512


def _dec_mlp_body(gc, gv, wa, wb, b0, w1, b1, wo, bo, o):
    gcv = jnp.concatenate([gc[0], gc[1]], axis=-1)
    gvv = jnp.concatenate([gv[0], gv[1]], axis=-1)
    u = (jnp.dot(gcv, wa[...], preferred_element_type=jnp.float32)
         + jnp.dot(gvv, wb[...], preferred_element_type=jnp.float32)
         + b0[...])
    u = jnp.where(u >= 0.0, u, 0.01 * u)
    v = jnp.dot(u, w1[...], preferred_element_type=jnp.float32) + b1[...]
    v = jnp.where(v >= 0.0, v, 0.01 * v)
    t = jnp.sum(v * wo[...], axis=1, keepdims=True) + bo[0, 0]
    p = jax.nn.sigmoid(t)
    o[...] = jnp.concatenate([p, 1.0 - p], axis=1)


_dec_mlp = pl.pallas_call(
    _dec_mlp_body,
    grid=(ELPAD // _DBM,),
    in_specs=[
        pl.BlockSpec((2, _DBM, HD), lambda i: (0, i, 0)),
        pl.BlockSpec((2, _DBM, HD), lambda i: (0, i, 0)),
        pl.BlockSpec((D, D), lambda i: (0, 0)),
        pl.BlockSpec((D, D), lambda i: (0, 0)),
        pl.BlockSpec((1, D), lambda i: (0, 0)),
        pl.BlockSpec((D, HD), lambda i: (0, 0)),
        pl.BlockSpec((1, HD), lambda i: (0, 0)),
        pl.BlockSpec((1, HD), lambda i: (0, 0)),
        pl.BlockSpec((1, 1), lambda i: (0, 0), memory_space=pltpu.SMEM),
    ],
    out_specs=pl.BlockSpec((_DBM, 2), lambda i: (i, 0)),
    out_shape=jax.ShapeDtypeStruct((ELPAD, 2), jnp.float32),
)


def _stack_halves(x):
    return jnp.concatenate([x[:, :HD], x[:, HD:]], axis=0)


def _prep_edges(ei):
    src = jnp.concatenate([ei[0], jnp.zeros((EPAD - E,), jnp.int32)])
    dst = jnp.concatenate([ei[1], jnp.full((EPAD - E,), N, jnp.int32)])
    srclo = src.reshape(EPAD // ECHUNK, ECHUNK)
    return srclo, srclo + N, dst.reshape(EPAD // ECHUNK, ECHUNK)


def kernel(x_customer, x_variant, edge_index_c2v, edge_index_v2c,
           edge_label_index, Wl1_c2v, Wr1_c2v, b1_c2v, Wl1_v2c, Wr1_v2c,
           b1_v2c, Wl2_c2v, Wr2_c2v, b2_c2v, Wl2_v2c, Wr2_v2c, b2_v2c,
           Wd0, bd0, Wd1, bd1, Wout, bout):
    f32 = jnp.float32
    xc2 = _stack_halves(x_customer)
    xv2 = _stack_halves(x_variant)
    c2v = _prep_edges(edge_index_c2v)
    v2c = _prep_edges(edge_index_v2c)

    z_a = jnp.zeros((ZR, HD), f32)
    z_c8 = jnp.zeros((ZR, CW), f32)
    o_c8 = jnp.ones((ECHUNK, CW), f32)

    cnt_c2v, cnt_v2c = _cnt(c2v[2], v2c[2], z_c8, o_c8)

    def _after(x, *deps):
        return lax.optimization_barrier(
            (x,) + tuple(d.ravel()[0] for d in deps))[0]

    s1v = _seg(_after(xc2, cnt_v2c), c2v[0], c2v[1], c2v[2], z_a)
    s1c = _seg(_after(xv2, s1v), v2c[0], v2c[1], v2c[2], z_a)
    h_v2 = _combine_l1(s1v.reshape(2, N, HD), cnt_c2v, x_variant,
                       Wl1_c2v.T, Wr1_c2v.T, b1_c2v.reshape(1, D))
    h_c2 = _combine_l1(s1c.reshape(2, N, HD), cnt_v2c, x_customer,
                       Wl1_v2c.T, Wr1_v2c.T, b1_v2c.reshape(1, D))

    s2c = _seg(_after(h_v2.reshape(NCORE * N, HD), s1c),
               v2c[0], v2c[1], v2c[2], z_a)
    s2v = _seg(_after(h_c2.reshape(NCORE * N, HD), s2c),
               c2v[0], c2v[1], c2v[2], z_a)
    z_c, z_c2 = _combine_l2(s2c.reshape(2, N, HD), cnt_v2c, h_c2,
                            Wl2_v2c.T, Wr2_v2c.T, b2_v2c.reshape(1, D))
    z_v, z_v2 = _combine_l2(s2v.reshape(2, N, HD), cnt_c2v, h_v2,
                            Wl2_c2v.T, Wr2_c2v.T, b2_c2v.reshape(1, D))

    rowlo = jnp.concatenate([edge_label_index[0],
                             jnp.zeros((ELPAD - EL,), jnp.int32)])
    collo = jnp.concatenate([edge_label_index[1],
                             jnp.zeros((ELPAD - EL,), jnp.int32)])
    rowlo = rowlo.reshape(ELPAD // ECHUNK, ECHUNK)
    collo = collo.reshape(ELPAD // ECHUNK, ECHUNK)
    gzc, gzv = _dec_gather(_after(z_c2.reshape(NCORE * N, HD), s2v),
                           z_v2.reshape(NCORE * N, HD),
                           rowlo, rowlo + N, collo, collo + N)
    scores = _dec_mlp(gzc.reshape(2, ELPAD, HD), gzv.reshape(2, ELPAD, HD),
                      Wd0[:, :D].T, Wd0[:, D:].T,
                      bd0.reshape(1, D), Wd1.T, bd1.reshape(1, HD),
                      Wout, bout.reshape(1, 1))
    return (scores[:EL], z_c, z_v)


# speedup vs baseline: 1.0504x; 1.0504x over previous
import functools

import jax
import jax.numpy as jnp
from jax import lax
from jax.experimental import pallas as pl
from jax.experimental.pallas import tpu as pltpu
from jax.experimental.pallas import tpu_sc as plsc

N = 10000
D = 256
HD = 128
E = 160000
EL = 50000

NSUB = 16
NCORE = 2

ECHUNK = 128
CPT = 80
EPAD = NSUB * CPT * ECHUNK
ACC_R = 10112
ZR = ACC_R // NSUB
OW = 624
OTAIL = N - NSUB * OW
CW = 128
CWI = 16

DCPT = 16
DW = 26
ELPAD = DW * DCPT * ECHUNK

_mesh = plsc.VectorSubcoreMesh(core_axis_name="c", subcore_axis_name="s")


HCPT = CPT // 2


def _seg_body(xt2, srclo, srchi, dstp, z_a, sums,
              acc, idxs, idxd, rows0, rows1, sem0, sem1):
    c = lax.axis_index("c")
    s = lax.axis_index("s")

    zoff = pl.multiple_of(s * ZR, 8)
    coff = pl.multiple_of(s * CPT, 8)
    pltpu.sync_copy(z_a, acc.at[pl.ds(zoff, ZR)])

    plsc.subcore_barrier()

    for h in range(2):
        pltpu.sync_copy(dstp.at[pl.ds(coff + h * HCPT, HCPT)], idxd)

        @pl.when(c == 0)
        def _():
            pltpu.sync_copy(srclo.at[pl.ds(coff + h * HCPT, HCPT)], idxs)

        @pl.when(c == 1)
        def _():
            pltpu.sync_copy(srchi.at[pl.ds(coff + h * HCPT, HCPT)], idxs)

        pltpu.async_copy(xt2.at[idxs.at[0]], rows0, sem0)

        def pair(g, carry):
            j0 = 2 * g
            j1 = j0 + 1
            pltpu.async_copy(xt2.at[idxs.at[j1]], rows1, sem1)
            pltpu.make_async_copy(xt2.at[idxs.at[j0]], rows0, sem0).wait()
            pltpu.sync_copy(rows0, acc.at[idxd.at[j0]], add=True)

            @pl.when(j1 + 1 < HCPT)
            def _():
                pltpu.async_copy(xt2.at[idxs.at[j1 + 1]], rows0, sem0)

            pltpu.make_async_copy(xt2.at[idxs.at[j1]], rows1, sem1).wait()
            pltpu.sync_copy(rows1, acc.at[idxd.at[j1]], add=True)
            return carry

        lax.fori_loop(0, HCPT // 2, pair, 0)

    plsc.subcore_barrier()

    pltpu.sync_copy(acc.at[pl.ds(pl.multiple_of(s * OW, 8), OW)],
                    sums.at[pl.ds(pl.multiple_of(c * N + s * OW, 8), OW)])

    @pl.when(s == NSUB - 1)
    def _():
        pltpu.sync_copy(
            acc.at[pl.ds(NSUB * OW, OTAIL)],
            sums.at[pl.ds(pl.multiple_of(c * N + NSUB * OW, 8), OTAIL)])


_seg = pl.kernel(
    _seg_body,
    out_type=jax.ShapeDtypeStruct((NCORE * N, HD), jnp.float32),
    mesh=_mesh,
    scratch_types=[
        pltpu.VMEM_SHARED((ACC_R, HD), jnp.float32),
        pltpu.VMEM((HCPT, ECHUNK), jnp.int32),
        pltpu.VMEM((HCPT, ECHUNK), jnp.int32),
        pltpu.VMEM((ECHUNK, HD), jnp.float32),
        pltpu.VMEM((ECHUNK, HD), jnp.float32),
        pltpu.SemaphoreType.DMA, pltpu.SemaphoreType.DMA,
    ],
)


def _cnt_body(dst_a, dst_b, z_c8, o_c8, cnt_a, cnt_b, cnta, idxd, onesb):
    c = lax.axis_index("c")
    s = lax.axis_index("s")
    zoff = pl.multiple_of(s * ZR, 8)
    coff = pl.multiple_of(s * CPT, 8)
    pltpu.sync_copy(z_c8, cnta.at[pl.ds(zoff, ZR)])
    pltpu.sync_copy(o_c8, onesb)

    plsc.subcore_barrier()

    for h in range(2):
        @pl.when(c == 0)
        def _():
            pltpu.sync_copy(dst_a.at[pl.ds(coff + h * HCPT, HCPT)], idxd)

        @pl.when(c == 1)
        def _():
            pltpu.sync_copy(dst_b.at[pl.ds(coff + h * HCPT, HCPT)], idxd)

        def jbody(j, carry):
            pltpu.sync_copy(onesb, cnta.at[idxd.at[j]], add=True)
            return carry

        lax.fori_loop(0, HCPT, jbody, 0)

    plsc.subcore_barrier()

    woff = pl.multiple_of(s * OW, 8)

    @pl.when(c == 0)
    def _():
        pltpu.sync_copy(cnta.at[pl.ds(woff, OW)], cnt_a.at[pl.ds(woff, OW)])

        @pl.when(s == NSUB - 1)
        def _():
            pltpu.sync_copy(cnta.at[pl.ds(NSUB * OW, OTAIL)],
                            cnt_a.at[pl.ds(NSUB * OW, OTAIL)])

    @pl.when(c == 1)
    def _():
        pltpu.sync_copy(cnta.at[pl.ds(woff, OW)], cnt_b.at[pl.ds(woff, OW)])

        @pl.when(s == NSUB - 1)
        def _():
            pltpu.sync_copy(cnta.at[pl.ds(NSUB * OW, OTAIL)],
                            cnt_b.at[pl.ds(NSUB * OW, OTAIL)])


_cnt = pl.kernel(
    _cnt_body,
    out_type=(jax.ShapeDtypeStruct((N, CW), jnp.float32),
              jax.ShapeDtypeStruct((N, CW), jnp.float32)),
    mesh=_mesh,
    scratch_types=[
        pltpu.VMEM_SHARED((ACC_R, CW), jnp.float32),
        pltpu.VMEM((HCPT, ECHUNK), jnp.int32),
        pltpu.VMEM((ECHUNK, CW), jnp.float32),
    ],
)


def _dec_gather_body(zc2, zv2, rowlo, rowhi, collo, colhi, gzc, gzv,
                     idxrl, idxrh, idxcl, idxch, bufa, bufb, bufc, bufd,
                     gsa, gsb, gsc, gsd, tsa, tsb, tsc, tsd):
    c = lax.axis_index("c")
    s = lax.axis_index("s")
    w = s * NCORE + c

    @pl.when(w < DW)
    def _():
        soff = pl.multiple_of(w * DCPT, 8)
        pltpu.sync_copy(rowlo.at[pl.ds(soff, DCPT)], idxrl)
        pltpu.sync_copy(rowhi.at[pl.ds(soff, DCPT)], idxrh)
        pltpu.sync_copy(collo.at[pl.ds(soff, DCPT)], idxcl)
        pltpu.sync_copy(colhi.at[pl.ds(soff, DCPT)], idxch)

        def stream(i):
            tbl = (zc2, zc2, zv2, zv2)[i]
            idx = (idxrl, idxrh, idxcl, idxch)[i]
            out = (gzc, gzc, gzv, gzv)[i]
            base = (0, ELPAD, 0, ELPAD)[i]
            buf = (bufa, bufb, bufc, bufd)[i]
            gs = (gsa, gsb, gsc, gsd)[i]
            ts = (tsa, tsb, tsc, tsd)[i]
            return tbl, idx, out, base, buf, gs, ts

        for i in range(4):
            tbl, idx, out, base, buf, gs, ts = stream(i)
            pltpu.async_copy(tbl.at[idx.at[0]], buf, gs)

        def jbody(j, carry):
            o = pl.multiple_of((w * DCPT + j) * ECHUNK, 8)
            for i in range(4):
                tbl, idx, out, base, buf, gs, ts = stream(i)
                pltpu.make_async_copy(tbl.at[idx.at[j]], buf, gs).wait()
                pltpu.async_copy(buf, out.at[pl.ds(base + o, ECHUNK)], ts)
            for i in range(4):
                tbl, idx, out, base, buf, gs, ts = stream(i)
                pltpu.make_async_copy(buf, out.at[pl.ds(base + o, ECHUNK)],
                                      ts).wait()

                @pl.when(j + 1 < DCPT)
                def _():
                    pltpu.async_copy(tbl.at[idx.at[j + 1]], buf, gs)

            return carry

        lax.fori_loop(0, DCPT, jbody, 0)


_dec_gather = pl.kernel(
    _dec_gather_body,
    out_type=(jax.ShapeDtypeStruct((2 * ELPAD, HD), jnp.float32),
              jax.ShapeDtypeStruct((2 * ELPAD, HD), jnp.float32)),
    mesh=_mesh,
    scratch_types=[
        pltpu.VMEM((DCPT, ECHUNK), jnp.int32),
        pltpu.VMEM((DCPT, ECHUNK), jnp.int32),
        pltpu.VMEM((DCPT, ECHUNK), jnp.int32),
        pltpu.VMEM((DCPT, ECHUNK), jnp.int32),
        pltpu.VMEM((ECHUNK, HD), jnp.float32),
        pltpu.VMEM((ECHUNK, HD), jnp.float32),
        pltpu.VMEM((ECHUNK, HD), jnp.float32),
        pltpu.VMEM((ECHUNK, HD), jnp.float32),
        pltpu.SemaphoreType.DMA, pltpu.SemaphoreType.DMA,
        pltpu.SemaphoreType.DMA, pltpu.SemaphoreType.DMA,
        pltpu.SemaphoreType.DMA, pltpu.SemaphoreType.DMA,
        pltpu.SemaphoreType.DMA, pltpu.SemaphoreType.DMA,
    ],
)


_BM = 1000


def _combine_body(s2, cntr, x, wl, wr, b, o, o2=None, *, relu, x_split,
                  o_split):
    s = jnp.concatenate([s2[0], s2[1]], axis=-1)
    mean = s / jnp.maximum(cntr[:, 0], 1.0)[:, None]
    if x_split:
        xv = jnp.concatenate([x[0], x[1]], axis=-1)
    else:
        xv = x[...]
    r = (jnp.dot(mean, wl[...], preferred_element_type=jnp.float32)
         + jnp.dot(xv, wr[...], preferred_element_type=jnp.float32)
         + b[...])
    if relu:
        r = jnp.maximum(r, 0.0)
    if o_split:
        o[0] = r[:, :HD]
        o[1] = r[:, HD:]
    else:
        o[...] = r
        o2[0] = r[:, :HD]
        o2[1] = r[:, HD:]


def _make_combine(relu, x_split, o_split):
    split3 = pl.BlockSpec((2, _BM, HD), lambda i: (0, i, 0))
    full2 = pl.BlockSpec((_BM, D), lambda i: (i, 0))
    wspec = pl.BlockSpec((D, D), lambda i: (0, 0))
    return pl.pallas_call(
        functools.partial(_combine_body, relu=relu, x_split=x_split,
                          o_split=o_split),
        grid=(N // _BM,),
        in_specs=[
            split3,
            pl.BlockSpec((_BM, CW), lambda i: (i, 0)),
            split3 if x_split else full2,
            wspec, wspec,
            pl.BlockSpec((1, D), lambda i: (0, 0)),
        ],
        out_specs=split3 if o_split else [full2, split3],
        out_shape=jax.ShapeDtypeStruct((2, N, HD), jnp.float32)
        if o_split else [jax.ShapeDtypeStruct((N, D), jnp.float32),
                         jax.ShapeDtypeStruct((2, N, HD), jnp.float32)],
    )


_combine_l1 = _make_combine(relu=True, x_split=False, o_split=True)
_combine_l2 = _make_combine(relu=False, x_split=True, o_split=False)


_DBM = 512


def _dec_mlp_body(gc, gv, wa, wb, b0, w1, b1, wo, bo, o):
    gcv = jnp.concatenate([gc[0], gc[1]], axis=-1)
    gvv = jnp.concatenate([gv[0], gv[1]], axis=-1)
    u = (jnp.dot(gcv, wa[...], preferred_element_type=jnp.float32)
         + jnp.dot(gvv, wb[...], preferred_element_type=jnp.float32)
         + b0[...])
    u = jnp.where(u >= 0.0, u, 0.01 * u)
    v = jnp.dot(u, w1[...], preferred_element_type=jnp.float32) + b1[...]
    v = jnp.where(v >= 0.0, v, 0.01 * v)
    t = jnp.sum(v * wo[...], axis=1, keepdims=True) + bo[0, 0]
    p = jax.nn.sigmoid(t)
    o[...] = jnp.concatenate([p, 1.0 - p], axis=1)


_dec_mlp = pl.pallas_call(
    _dec_mlp_body,
    grid=(ELPAD // _DBM,),
    in_specs=[
        pl.BlockSpec((2, _DBM, HD), lambda i: (0, i, 0)),
        pl.BlockSpec((2, _DBM, HD), lambda i: (0, i, 0)),
        pl.BlockSpec((D, D), lambda i: (0, 0)),
        pl.BlockSpec((D, D), lambda i: (0, 0)),
        pl.BlockSpec((1, D), lambda i: (0, 0)),
        pl.BlockSpec((D, HD), lambda i: (0, 0)),
        pl.BlockSpec((1, HD), lambda i: (0, 0)),
        pl.BlockSpec((1, HD), lambda i: (0, 0)),
        pl.BlockSpec((1, 1), lambda i: (0, 0), memory_space=pltpu.SMEM),
    ],
    out_specs=pl.BlockSpec((_DBM, 2), lambda i: (i, 0)),
    out_shape=jax.ShapeDtypeStruct((ELPAD, 2), jnp.float32),
)


def _stack_halves(x):
    return jnp.concatenate([x[:, :HD], x[:, HD:]], axis=0)


def _prep_edges(ei):
    src = jnp.concatenate([ei[0], jnp.zeros((EPAD - E,), jnp.int32)])
    dst = jnp.concatenate([ei[1], jnp.full((EPAD - E,), N, jnp.int32)])
    srclo = src.reshape(EPAD // ECHUNK, ECHUNK)
    return srclo, srclo + N, dst.reshape(EPAD // ECHUNK, ECHUNK)


def kernel(x_customer, x_variant, edge_index_c2v, edge_index_v2c,
           edge_label_index, Wl1_c2v, Wr1_c2v, b1_c2v, Wl1_v2c, Wr1_v2c,
           b1_v2c, Wl2_c2v, Wr2_c2v, b2_c2v, Wl2_v2c, Wr2_v2c, b2_v2c,
           Wd0, bd0, Wd1, bd1, Wout, bout):
    f32 = jnp.float32
    xc2 = _stack_halves(x_customer)
    xv2 = _stack_halves(x_variant)
    c2v = _prep_edges(edge_index_c2v)
    v2c = _prep_edges(edge_index_v2c)

    z_a = jnp.zeros((ZR, HD), f32)
    z_c8 = jnp.zeros((ZR, CW), f32)
    o_c8 = jnp.ones((ECHUNK, CW), f32)

    cnt_c2v, cnt_v2c = _cnt(c2v[2], v2c[2], z_c8, o_c8)

    def _after(x, *deps):
        return lax.optimization_barrier(
            (x,) + tuple(d.ravel()[0] for d in deps))[0]

    s1v = _seg(_after(xc2, cnt_v2c), c2v[0], c2v[1], c2v[2], z_a)
    s1c = _seg(_after(xv2, s1v), v2c[0], v2c[1], v2c[2], z_a)
    h_v2 = _combine_l1(s1v.reshape(2, N, HD), cnt_c2v, x_variant,
                       Wl1_c2v.T, Wr1_c2v.T, b1_c2v.reshape(1, D))
    h_c2 = _combine_l1(s1c.reshape(2, N, HD), cnt_v2c, x_customer,
                       Wl1_v2c.T, Wr1_v2c.T, b1_v2c.reshape(1, D))

    s2c = _seg(_after(h_v2.reshape(NCORE * N, HD), s1c),
               v2c[0], v2c[1], v2c[2], z_a)
    s2v = _seg(_after(h_c2.reshape(NCORE * N, HD), s2c),
               c2v[0], c2v[1], c2v[2], z_a)
    z_c, z_c2 = _combine_l2(s2c.reshape(2, N, HD), cnt_v2c, h_c2,
                            Wl2_v2c.T, Wr2_v2c.T, b2_v2c.reshape(1, D))
    z_v, z_v2 = _combine_l2(s2v.reshape(2, N, HD), cnt_c2v, h_v2,
                            Wl2_c2v.T, Wr2_c2v.T, b2_c2v.reshape(1, D))

    rowlo = jnp.concatenate([edge_label_index[0],
                             jnp.zeros((ELPAD - EL,), jnp.int32)])
    collo = jnp.concatenate([edge_label_index[1],
                             jnp.zeros((ELPAD - EL,), jnp.int32)])
    rowlo = rowlo.reshape(ELPAD // ECHUNK, ECHUNK)
    collo = collo.reshape(ELPAD // ECHUNK, ECHUNK)
    gzc, gzv = _dec_gather(_after(z_c2.reshape(NCORE * N, HD), s2v),
                           z_v2.reshape(NCORE * N, HD),
                           rowlo, rowlo + N, collo, collo + N)
    scores = _dec_mlp(gzc.reshape(2, ELPAD, HD), gzv.reshape(2, ELPAD, HD),
                      Wd0[:, :D].T, Wd0[:, D:].T,
                      bd0.reshape(1, D), Wd1.T, bd1.reshape(1, HD),
                      Wout, bout.reshape(1, 1))
    return (scores[:EL], z_c, z_v)
